# Initial kernel scaffold; baseline (speedup 1.0000x reference)
import jax
import jax.numpy as jnp
from jax.experimental import pallas as pl


def kernel(pos1, pos2, feature1, feature2, W0, W1, W2, g0, b0, g1, b1, g2, b2):
    raise NotImplementedError("write your pallas kernel here")


# speedup vs baseline: 7.5869x; 7.5869x over previous
import functools

import jax
import jax.numpy as jnp
from jax import lax
from jax.experimental import pallas as pl
from jax.experimental.pallas import tpu as pltpu
from jax.experimental.pallas import tpu_sc as plsc

B, N, C, NS = 4, 2048, 128, 16
P = B * N
M = NS * P
EPS = 1e-5

_INTERP = False


def _proj_body(pos1_ref, pos2_ref, f1_ref, f2_ref, wp_ref, wf1_ref, wf2_ref,
               g_ref, h_ref):
    dn = (((0,), (1,)), ((), ()))
    g = lax.dot_general(f2_ref[0], wf2_ref[...], dn,
                        preferred_element_type=jnp.float32)
    g += lax.dot_general(pos2_ref[0], wp_ref[...], dn,
                         preferred_element_type=jnp.float32)
    g_ref[...] = g
    h = lax.dot_general(f1_ref[0], wf1_ref[...], dn,
                        preferred_element_type=jnp.float32)
    h -= lax.dot_general(pos1_ref[0], wp_ref[...], dn,
                         preferred_element_type=jnp.float32)
    h_ref[...] = h


def _proj(pos1, pos2, f1, f2, wp, wf1, wf2):
    return pl.pallas_call(
        _proj_body,
        grid=(B,),
        in_specs=[
            pl.BlockSpec((1, 3, N), lambda b: (b, 0, 0)),
            pl.BlockSpec((1, 3, N), lambda b: (b, 0, 0)),
            pl.BlockSpec((1, C, N), lambda b: (b, 0, 0)),
            pl.BlockSpec((1, C, N), lambda b: (b, 0, 0)),
            pl.BlockSpec((C, 3), lambda b: (0, 0)),
            pl.BlockSpec((C, C), lambda b: (0, 0)),
            pl.BlockSpec((C, C), lambda b: (0, 0)),
        ],
        out_specs=[
            pl.BlockSpec((N, C), lambda b: (b, 0)),
            pl.BlockSpec((N, C), lambda b: (b, 0)),
        ],
        out_shape=[
            jax.ShapeDtypeStruct((P, C), jnp.float32),
            jax.ShapeDtypeStruct((P, C), jnp.float32),
        ],
        interpret=_INTERP,
    )(pos1, pos2, f1, f2, wp, wf1, wf2)


_RB = 256


def _knn_body(p1_ref, p2_ref, out_ref):
    b = pl.program_id(0)
    p1 = p1_ref[0]
    p2 = p2_ref[0]
    d = -2.0 * lax.dot_general(p1, p2, (((0,), (0,)), ((), ())),
                               preferred_element_type=jnp.float32)
    d += jnp.sum(p1 * p1, axis=0)[:, None]
    d += jnp.sum(p2 * p2, axis=0)[None, :]
    iota = lax.broadcasted_iota(jnp.int32, (_RB, N), 1)
    inf = jnp.float32(jnp.inf)
    for s in range(NS):
        m = jnp.min(d, axis=1)
        am = jnp.min(jnp.where(d == m[:, None], iota, N), axis=1)
        out_ref[s, :] = am + b * N
        d = jnp.where(iota == am[:, None], inf, d)


def _knn(pos1, pos2):
    return pl.pallas_call(
        _knn_body,
        grid=(B, N // _RB),
        in_specs=[
            pl.BlockSpec((1, 3, _RB), lambda b, i: (b, 0, i)),
            pl.BlockSpec((1, 3, N), lambda b, i: (b, 0, 0)),
        ],
        out_specs=pl.BlockSpec((NS, _RB), lambda b, i: (0, b * (N // _RB) + i)),
        out_shape=jax.ShapeDtypeStruct((NS, P), jnp.int32),
        interpret=_INTERP,
    )(pos1, pos2)


_NC_SC, _NSUB_SC = 2, 16
_NW = _NC_SC * _NSUB_SC
_ROWS_W = M // _NW
_CHUNK = 128
_NCHUNK = _ROWS_W // _CHUNK


def _gather_sc(table, idx2d):
    mesh = plsc.VectorSubcoreMesh(core_axis_name="c", subcore_axis_name="s")

    @functools.partial(
        pl.kernel, mesh=mesh,
        out_type=jax.ShapeDtypeStruct((M, C), jnp.float32),
        scratch_types=[
            pltpu.VMEM((_NCHUNK, _CHUNK), jnp.int32),
            pltpu.VMEM((_CHUNK, C), jnp.float32),
            pltpu.VMEM((_CHUNK, C), jnp.float32),
            pltpu.SemaphoreType.DMA,
            pltpu.SemaphoreType.DMA,
        ],
    )
    def k(table_hbm, idx_hbm, out_hbm, idx_v, buf0, buf1, sem0, sem1):
        wid = lax.axis_index("s") * _NC_SC + lax.axis_index("c")
        pltpu.sync_copy(idx_hbm.at[pl.ds(wid * _NCHUNK, _NCHUNK)], idx_v)
        out_base = wid * _ROWS_W

        def body(j2, _):
            j0 = j2 * 2
            cp0 = pltpu.async_copy(table_hbm.at[idx_v.at[j0]], buf0, sem0)
            cp1 = pltpu.async_copy(table_hbm.at[idx_v.at[j0 + 1]], buf1, sem1)
            cp0.wait()
            pltpu.sync_copy(buf0, out_hbm.at[pl.ds(out_base + j0 * _CHUNK,
                                                   _CHUNK)])
            cp1.wait()
            pltpu.sync_copy(buf1, out_hbm.at[pl.ds(out_base + (j0 + 1) * _CHUNK,
                                                   _CHUNK)])
            return 0

        lax.fori_loop(0, _NCHUNK // 2, body, 0)

    return k(table, idx2d)


_PB = 1024


def _stats_body(gg_ref, ht_ref, sum_ref, sq_ref):
    @pl.when((pl.program_id(0) == 0) & (pl.program_id(1) == 0))
    def _():
        sum_ref[...] = jnp.zeros_like(sum_ref)
        sq_ref[...] = jnp.zeros_like(sq_ref)

    y = gg_ref[0] + ht_ref[...]
    sum_ref[0, :] += jnp.sum(y, axis=0)
    sq_ref[0, :] += jnp.sum(y * y, axis=0)


def _stats1(gg, ht):
    return pl.pallas_call(
        _stats_body,
        grid=(P // _PB, NS),
        in_specs=[
            pl.BlockSpec((1, _PB, C), lambda i, s: (s, i, 0)),
            pl.BlockSpec((_PB, C), lambda i, s: (i, 0)),
        ],
        out_specs=[
            pl.BlockSpec((1, C), lambda i, s: (0, 0)),
            pl.BlockSpec((1, C), lambda i, s: (0, 0)),
        ],
        out_shape=[
            jax.ShapeDtypeStruct((1, C), jnp.float32),
            jax.ShapeDtypeStruct((1, C), jnp.float32),
        ],
        interpret=_INTERP,
    )(gg.reshape(NS, P, C), ht)


def _affine(g, b, s, q):
    mean = s / jnp.float32(M)
    var = q / jnp.float32(M) - mean * mean
    a = g * lax.rsqrt(var + EPS)
    c = b - mean * a
    return a, c


def _layer_ht_body(gg_ref, ht_ref, w_ref, g_ref, b_ref, s_ref, q_ref,
                   out_ref, sum_ref, sq_ref):
    @pl.when((pl.program_id(0) == 0) & (pl.program_id(1) == 0))
    def _():
        sum_ref[...] = jnp.zeros_like(sum_ref)
        sq_ref[...] = jnp.zeros_like(sq_ref)

    a, c = _affine(g_ref[0, :], b_ref[0, :], s_ref[0, :], q_ref[0, :])
    y = gg_ref[0] + ht_ref[...]
    x = jnp.maximum(y * a[None, :] + c[None, :], 0.0)
    out = lax.dot_general(x, w_ref[...], (((1,), (1,)), ((), ())),
                          preferred_element_type=jnp.float32)
    out_ref[0] = out
    sum_ref[0, :] += jnp.sum(out, axis=0)
    sq_ref[0, :] += jnp.sum(out * out, axis=0)


def _layer_plain_body(y_ref, w_ref, g_ref, b_ref, s_ref, q_ref,
                      out_ref, sum_ref, sq_ref):
    @pl.when((pl.program_id(0) == 0) & (pl.program_id(1) == 0))
    def _():
        sum_ref[...] = jnp.zeros_like(sum_ref)
        sq_ref[...] = jnp.zeros_like(sq_ref)

    a, c = _affine(g_ref[0, :], b_ref[0, :], s_ref[0, :], q_ref[0, :])
    x = jnp.maximum(y_ref[0] * a[None, :] + c[None, :], 0.0)
    out = lax.dot_general(x, w_ref[...], (((1,), (1,)), ((), ())),
                          preferred_element_type=jnp.float32)
    out_ref[0] = out
    sum_ref[0, :] += jnp.sum(out, axis=0)
    sq_ref[0, :] += jnp.sum(out * out, axis=0)


_SMALL = [pl.BlockSpec((C, C), lambda i, s: (0, 0))] + \
         [pl.BlockSpec((1, C), lambda i, s: (0, 0))] * 4

_LAYER_OUT_SPECS = [
    pl.BlockSpec((1, _PB, C), lambda i, s: (s, i, 0)),
    pl.BlockSpec((1, C), lambda i, s: (0, 0)),
    pl.BlockSpec((1, C), lambda i, s: (0, 0)),
]
_LAYER_OUT_SHAPE = [
    jax.ShapeDtypeStruct((NS, P, C), jnp.float32),
    jax.ShapeDtypeStruct((1, C), jnp.float32),
    jax.ShapeDtypeStruct((1, C), jnp.float32),
]


def _layer_ht(gg, ht, w, g, b, s, q):
    return pl.pallas_call(
        _layer_ht_body,
        grid=(P // _PB, NS),
        in_specs=[
            pl.BlockSpec((1, _PB, C), lambda i, s: (s, i, 0)),
            pl.BlockSpec((_PB, C), lambda i, s: (i, 0)),
        ] + _SMALL,
        out_specs=_LAYER_OUT_SPECS,
        out_shape=_LAYER_OUT_SHAPE,
        interpret=_INTERP,
    )(gg.reshape(NS, P, C), ht, w, g, b, s, q)


def _layer_plain(y, w, g, b, s, q):
    return pl.pallas_call(
        _layer_plain_body,
        grid=(P // _PB, NS),
        in_specs=[pl.BlockSpec((1, _PB, C), lambda i, s: (s, i, 0))] + _SMALL,
        out_specs=_LAYER_OUT_SPECS,
        out_shape=_LAYER_OUT_SHAPE,
        interpret=_INTERP,
    )(y, w, g, b, s, q)


_PB2 = 512


def _final_body(y_ref, g_ref, b_ref, s_ref, q_ref, out_ref):
    a, c = _affine(g_ref[0, :], b_ref[0, :], s_ref[0, :], q_ref[0, :])
    x = jnp.maximum(y_ref[...] * a[None, None, :] + c[None, None, :], 0.0)
    r = jnp.max(x, axis=0)
    out_ref[0] = r.T


def _final(y, g, b, s, q):
    nb = N // _PB2
    return pl.pallas_call(
        _final_body,
        grid=(P // _PB2,),
        in_specs=[
            pl.BlockSpec((NS, _PB2, C), lambda t: (0, t, 0)),
            pl.BlockSpec((1, C), lambda t: (0, 0)),
            pl.BlockSpec((1, C), lambda t: (0, 0)),
            pl.BlockSpec((1, C), lambda t: (0, 0)),
            pl.BlockSpec((1, C), lambda t: (0, 0)),
        ],
        out_specs=pl.BlockSpec((1, C, _PB2), lambda t: (t // nb, 0, t % nb)),
        out_shape=jax.ShapeDtypeStruct((B, C, N), jnp.float32),
        interpret=_INTERP,
    )(y, g, b, s, q)


def kernel(pos1, pos2, feature1, feature2, W0, W1, W2, g0, b0, g1, b1, g2, b2):
    wp = W0[:, :3]
    wf2 = W0[:, 3:3 + C]
    wf1 = W0[:, 3 + C:]
    r = lambda v: v.reshape(1, C)

    gt, ht = _proj(pos1, pos2, feature1, feature2, wp, wf1, wf2)
    idxf = _knn(pos1, pos2)
    gg = _gather_sc(gt, idxf.reshape(M // _CHUNK, _CHUNK))
    s1, q1 = _stats1(gg, ht)
    y2, s2, q2 = _layer_ht(gg, ht, W1, r(g0), r(b0), s1, q1)
    y3, s3, q3 = _layer_plain(y2, W2, r(g1), r(b1), s2, q2)
    feat = _final(y3, r(g2), r(b2), s3, q3)
    return (pos1, feat)
